# baseline (speedup 1.0000x reference)
import functools

import jax
import jax.numpy as jnp
from jax import lax
from jax.experimental import pallas as pl
from jax.experimental.pallas import tpu as pltpu
from jax.experimental.pallas import tpu_sc as plsc

BATCH = 16384
NROWS = 1000000
EMB = 64
HID = 128

_NC = 2
_NS = 16
_NW = _NC * _NS
_RPW = BATCH // _NW
_CH = 128
_NCHUNK = _RPW // _CH

_BT = 8192
_BB = 2048


def _round_bits(x):
    bits = lax.bitcast_convert_type(x, jnp.uint32)
    rounded = bits + 0x7FFF + ((bits >> 16) & 1)
    return rounded & jnp.uint32(0xFFFF0000)


def _pack_body(a_ref, b_ref, eye_ref, out_ref):
    tr = lambda x: lax.dot_general(
        x, eye_ref[...], (((0,), (0,)), ((), ())),
        preferred_element_type=jnp.float32)
    hi = _round_bits(tr(a_ref[...]))
    lo = _round_bits(tr(b_ref[...])) >> 16
    out_ref[...] = lax.bitcast_convert_type(hi | lo, jnp.float32)


def _pack(tA, tB, eye):
    grid = pl.cdiv(NROWS, _BT)
    return pl.pallas_call(
        _pack_body,
        grid=(grid,),
        in_specs=[pl.BlockSpec((EMB, _BT), lambda i: (0, i)),
                  pl.BlockSpec((EMB, _BT), lambda i: (0, i)),
                  pl.BlockSpec((EMB, EMB), lambda i: (0, 0))],
        out_specs=pl.BlockSpec((_BT, EMB), lambda i: (i, 0)),
        out_shape=jax.ShapeDtypeStruct((NROWS, EMB), jnp.float32),
        compiler_params=pltpu.CompilerParams(
            dimension_semantics=("arbitrary",),
            vmem_limit_bytes=100 * 1024 * 1024),
    )(tA, tB, eye)


def _sc_gather_body(uidx_hbm, iidx_hbm, pu_hbm, pi_hbm,
                    out_u, out_i,
                    uidx_v, iidx_v, buf_a, buf_b, sem_a, sem_b):
    wid = lax.axis_index("s") * _NC + lax.axis_index("c")
    base = wid * _RPW
    pltpu.sync_copy(uidx_hbm.at[pl.ds(wid * _NCHUNK, _NCHUNK)], uidx_v)
    pltpu.sync_copy(iidx_hbm.at[pl.ds(wid * _NCHUNK, _NCHUNK)], iidx_v)

    bufs = (buf_a, buf_b)
    sems = (sem_a, sem_b)

    for tbl, idxv, out in ((pu_hbm, uidx_v, out_u), (pi_hbm, iidx_v, out_i)):
        def fire(c):
            return pltpu.async_copy(tbl.at[idxv.at[c]], bufs[c % 2],
                                    sems[c % 2])

        pending = fire(0)
        for c in range(_NCHUNK):
            nxt = fire(c + 1) if c + 1 < _NCHUNK else None
            pending.wait()
            pltpu.sync_copy(bufs[c % 2],
                            out.at[pl.ds(base + c * _CH, _CH)])
            pending = nxt


@functools.cache
def _sc_gather():
    return pl.kernel(
        _sc_gather_body,
        mesh=plsc.VectorSubcoreMesh(core_axis_name="c", subcore_axis_name="s"),
        out_type=[jax.ShapeDtypeStruct((BATCH, EMB), jnp.float32)] * 2,
        scratch_types=[
            pltpu.VMEM((_NCHUNK, _CH), jnp.int32),
            pltpu.VMEM((_NCHUNK, _CH), jnp.int32),
            pltpu.VMEM((_CH, EMB), jnp.float32),
            pltpu.VMEM((_CH, EMB), jnp.float32),
            pltpu.SemaphoreType.DMA,
            pltpu.SemaphoreType.DMA,
        ],
        compiler_params=pltpu.CompilerParams(use_tc_tiling_on_sc=False),
    )


def _unpack(x):
    bits = lax.bitcast_convert_type(x, jnp.uint32)
    hi = lax.bitcast_convert_type(bits & jnp.uint32(0xFFFF0000), jnp.float32)
    lo = lax.bitcast_convert_type(bits << 16, jnp.float32)
    return hi, lo


def _mlp_body(gu_ref, gi_ref,
              w1a_ref, w1b_ref, b1_ref, w2_ref, b2_ref, w3_ref, b3_ref,
              wog_ref, woh_ref, bo_ref, out_ref):
    dot = functools.partial(jnp.dot, preferred_element_type=jnp.float32)
    gmf_u, mlp_u = _unpack(gu_ref[...])
    gmf_i, mlp_i = _unpack(gi_ref[...])
    h = jnp.maximum(dot(mlp_u, w1a_ref[...]) +
                    dot(mlp_i, w1b_ref[...]) + b1_ref[...], 0.0)
    h = jnp.maximum(dot(h, w2_ref[...]) + b2_ref[...], 0.0)
    h = jnp.maximum(dot(h, w3_ref[...]) + b3_ref[...], 0.0)
    gmf = gmf_u * gmf_i
    out_ref[...] = dot(gmf, wog_ref[...]) + dot(h, woh_ref[...]) + bo_ref[...]


def _mlp(gu, gi, w1a, w1b, b1, w2, b2, w3, b3, wog, woh, bo):
    grid = BATCH // _BB
    row = lambda i: (i, 0)
    rep = lambda i: (0, 0)
    emb_spec = pl.BlockSpec((_BB, EMB), row)
    full = lambda a: pl.BlockSpec(a.shape, rep)
    return pl.pallas_call(
        _mlp_body,
        grid=(grid,),
        in_specs=[emb_spec, emb_spec,
                  full(w1a), full(w1b), full(b1), full(w2), full(b2),
                  full(w3), full(b3), full(wog), full(woh), full(bo)],
        out_specs=pl.BlockSpec((_BB, 1), row),
        out_shape=jax.ShapeDtypeStruct((BATCH, 1), jnp.float32),
        compiler_params=pltpu.CompilerParams(
            dimension_semantics=("arbitrary",)),
    )(gu, gi, w1a, w1b, b1, w2, b2, w3, b3, wog, woh, bo)


def kernel(user, item, gmf_user, gmf_item, mlp_user, mlp_item,
           W1, b1, W2, b2, W3, b3, Wo, bo):
    user2d = user.astype(jnp.int32).reshape(BATCH // _CH, _CH)
    item2d = item.astype(jnp.int32).reshape(BATCH // _CH, _CH)
    eye = jnp.eye(EMB, dtype=jnp.float32)
    p_u = _pack(gmf_user.T, mlp_user.T, eye)
    p_i = _pack(gmf_item.T, mlp_item.T, eye)
    g_u, g_i = _sc_gather()(user2d, item2d, p_u, p_i)
    out = _mlp(g_u, g_i,
               W1[:EMB], W1[EMB:], b1.reshape(1, HID),
               W2, b2.reshape(1, HID // 2), W3, b3.reshape(1, EMB),
               Wo[:EMB], Wo[EMB:], bo.reshape(1, 1))
    return out.reshape(BATCH)


# speedup vs baseline: 1.0079x; 1.0079x over previous
import functools

import jax
import jax.numpy as jnp
from jax import lax
from jax.experimental import pallas as pl
from jax.experimental.pallas import tpu as pltpu
from jax.experimental.pallas import tpu_sc as plsc

BATCH = 16384
NROWS = 1000000
EMB = 64
HID = 128

_NC = 2
_NS = 16
_NW = _NC * _NS
_RPW = BATCH // _NW
_CH = 128
_NCHUNK = _RPW // _CH

_BT = 8192
_BB = 2048


def _round_bits(x):
    bits = lax.bitcast_convert_type(x, jnp.uint32)
    rounded = bits + 0x7FFF + ((bits >> 16) & 1)
    return rounded & jnp.uint32(0xFFFF0000)


def _pack_body(a_ref, b_ref, out_ref):
    hi = _round_bits(a_ref[...].T)
    lo = _round_bits(b_ref[...].T) >> 16
    out_ref[...] = lax.bitcast_convert_type(hi | lo, jnp.float32)


def _pack(tA, tB):
    grid = pl.cdiv(NROWS, _BT)
    return pl.pallas_call(
        _pack_body,
        grid=(grid,),
        in_specs=[pl.BlockSpec((EMB, _BT), lambda i: (0, i)),
                  pl.BlockSpec((EMB, _BT), lambda i: (0, i))],
        out_specs=pl.BlockSpec((_BT, EMB), lambda i: (i, 0)),
        out_shape=jax.ShapeDtypeStruct((NROWS, EMB), jnp.float32),
        compiler_params=pltpu.CompilerParams(
            dimension_semantics=("arbitrary",),
            vmem_limit_bytes=100 * 1024 * 1024),
    )(tA, tB)


def _sc_gather_body(uidx_hbm, iidx_hbm, pu_hbm, pi_hbm,
                    out_u, out_i,
                    uidx_v, iidx_v, buf_a, buf_b, sem_a, sem_b):
    wid = lax.axis_index("s") * _NC + lax.axis_index("c")
    base = wid * _RPW
    pltpu.sync_copy(uidx_hbm.at[pl.ds(wid * _NCHUNK, _NCHUNK)], uidx_v)
    pltpu.sync_copy(iidx_hbm.at[pl.ds(wid * _NCHUNK, _NCHUNK)], iidx_v)

    bufs = (buf_a, buf_b)
    sems = (sem_a, sem_b)

    for tbl, idxv, out in ((pu_hbm, uidx_v, out_u), (pi_hbm, iidx_v, out_i)):
        def fire(c):
            return pltpu.async_copy(tbl.at[idxv.at[c]], bufs[c % 2],
                                    sems[c % 2])

        pending = fire(0)
        for c in range(_NCHUNK):
            nxt = fire(c + 1) if c + 1 < _NCHUNK else None
            pending.wait()
            pltpu.sync_copy(bufs[c % 2],
                            out.at[pl.ds(base + c * _CH, _CH)])
            pending = nxt


@functools.cache
def _sc_gather():
    return pl.kernel(
        _sc_gather_body,
        mesh=plsc.VectorSubcoreMesh(core_axis_name="c", subcore_axis_name="s"),
        out_type=[jax.ShapeDtypeStruct((BATCH, EMB), jnp.float32)] * 2,
        scratch_types=[
            pltpu.VMEM((_NCHUNK, _CH), jnp.int32),
            pltpu.VMEM((_NCHUNK, _CH), jnp.int32),
            pltpu.VMEM((_CH, EMB), jnp.float32),
            pltpu.VMEM((_CH, EMB), jnp.float32),
            pltpu.SemaphoreType.DMA,
            pltpu.SemaphoreType.DMA,
        ],
        compiler_params=pltpu.CompilerParams(use_tc_tiling_on_sc=False),
    )


def _unpack(x):
    bits = lax.bitcast_convert_type(x, jnp.uint32)
    hi = lax.bitcast_convert_type(bits & jnp.uint32(0xFFFF0000), jnp.float32)
    lo = lax.bitcast_convert_type(bits << 16, jnp.float32)
    return hi, lo


def _mlp_body(gu_ref, gi_ref,
              w1a_ref, w1b_ref, b1_ref, w2_ref, b2_ref, w3_ref, b3_ref,
              wog_ref, woh_ref, bo_ref, out_ref):
    dot = functools.partial(jnp.dot, preferred_element_type=jnp.float32)
    gmf_u, mlp_u = _unpack(gu_ref[...])
    gmf_i, mlp_i = _unpack(gi_ref[...])
    h = jnp.maximum(dot(mlp_u, w1a_ref[...]) +
                    dot(mlp_i, w1b_ref[...]) + b1_ref[...], 0.0)
    h = jnp.maximum(dot(h, w2_ref[...]) + b2_ref[...], 0.0)
    h = jnp.maximum(dot(h, w3_ref[...]) + b3_ref[...], 0.0)
    gmf = gmf_u * gmf_i
    out_ref[...] = dot(gmf, wog_ref[...]) + dot(h, woh_ref[...]) + bo_ref[...]


def _mlp(gu, gi, w1a, w1b, b1, w2, b2, w3, b3, wog, woh, bo):
    grid = BATCH // _BB
    row = lambda i: (i, 0)
    rep = lambda i: (0, 0)
    emb_spec = pl.BlockSpec((_BB, EMB), row)
    full = lambda a: pl.BlockSpec(a.shape, rep)
    return pl.pallas_call(
        _mlp_body,
        grid=(grid,),
        in_specs=[emb_spec, emb_spec,
                  full(w1a), full(w1b), full(b1), full(w2), full(b2),
                  full(w3), full(b3), full(wog), full(woh), full(bo)],
        out_specs=pl.BlockSpec((_BB, 1), row),
        out_shape=jax.ShapeDtypeStruct((BATCH, 1), jnp.float32),
        compiler_params=pltpu.CompilerParams(
            dimension_semantics=("arbitrary",)),
    )(gu, gi, w1a, w1b, b1, w2, b2, w3, b3, wog, woh, bo)


def kernel(user, item, gmf_user, gmf_item, mlp_user, mlp_item,
           W1, b1, W2, b2, W3, b3, Wo, bo):
    user2d = user.astype(jnp.int32).reshape(BATCH // _CH, _CH)
    item2d = item.astype(jnp.int32).reshape(BATCH // _CH, _CH)
    p_u = _pack(gmf_user.T, mlp_user.T)
    p_i = _pack(gmf_item.T, mlp_item.T)
    g_u, g_i = _sc_gather()(user2d, item2d, p_u, p_i)
    out = _mlp(g_u, g_i,
               W1[:EMB], W1[EMB:], b1.reshape(1, HID),
               W2, b2.reshape(1, HID // 2), W3, b3.reshape(1, EMB),
               Wo[:EMB], Wo[EMB:], bo.reshape(1, 1))
    return out.reshape(BATCH)


# speedup vs baseline: 2.1157x; 2.0991x over previous
import functools

import jax
import jax.numpy as jnp
from jax import lax
from jax.experimental import pallas as pl
from jax.experimental.pallas import tpu as pltpu
from jax.experimental.pallas import tpu_sc as plsc

BATCH = 16384
NROWS = 1000000
EMB = 64
HID = 128

_NC = 2
_NS = 16
_NW = _NC * _NS
_RPW = BATCH // _NW
_CH = 128
_NCHUNK = _RPW // _CH

_BT = 8192
_BB = 2048


def _round_bits(x):
    bits = lax.bitcast_convert_type(x, jnp.uint32)
    rounded = bits + 0x7FFF + ((bits >> 16) & 1)
    return rounded & jnp.uint32(0xFFFF0000)


def _pack_body(a_ref, b_ref, c_ref, d_ref, out_ref):
    pair = lambda x_ref, y_ref: (_round_bits(x_ref[...].T) |
                                 (_round_bits(y_ref[...].T) >> 16))
    out_ref[...] = lax.bitcast_convert_type(
        jnp.concatenate([pair(a_ref, b_ref), pair(c_ref, d_ref)], axis=1),
        jnp.float32)


def _pack(tA, tB, tC, tD):
    grid = pl.cdiv(NROWS, _BT)
    tspec = pl.BlockSpec((EMB, _BT), lambda i: (0, i))
    return pl.pallas_call(
        _pack_body,
        grid=(grid,),
        in_specs=[tspec, tspec, tspec, tspec],
        out_specs=pl.BlockSpec((_BT, 2 * EMB), lambda i: (i, 0)),
        out_shape=jax.ShapeDtypeStruct((NROWS, 2 * EMB), jnp.float32),
        compiler_params=pltpu.CompilerParams(
            dimension_semantics=("arbitrary",),
            vmem_limit_bytes=100 * 1024 * 1024),
    )(tA, tB, tC, tD)


def _sc_gather_body(uidx_hbm, iidx_hbm, p_hbm,
                    out_u, out_i,
                    uidx_v, iidx_v, buf_a, buf_b, sem_a, sem_b):
    wid = lax.axis_index("s") * _NC + lax.axis_index("c")
    base = wid * _RPW
    pltpu.sync_copy(uidx_hbm.at[pl.ds(wid * _NCHUNK, _NCHUNK)], uidx_v)
    pltpu.sync_copy(iidx_hbm.at[pl.ds(wid * _NCHUNK, _NCHUNK)], iidx_v)

    bufs = (buf_a, buf_b)
    sems = (sem_a, sem_b)

    for tbl, idxv, out in ((p_hbm, uidx_v, out_u), (p_hbm, iidx_v, out_i)):
        def fire(c):
            return pltpu.async_copy(tbl.at[idxv.at[c]], bufs[c % 2],
                                    sems[c % 2])

        pending = fire(0)
        for c in range(_NCHUNK):
            nxt = fire(c + 1) if c + 1 < _NCHUNK else None
            pending.wait()
            pltpu.sync_copy(bufs[c % 2],
                            out.at[pl.ds(base + c * _CH, _CH)])
            pending = nxt


@functools.cache
def _sc_gather():
    return pl.kernel(
        _sc_gather_body,
        mesh=plsc.VectorSubcoreMesh(core_axis_name="c", subcore_axis_name="s"),
        out_type=[jax.ShapeDtypeStruct((BATCH, 2 * EMB), jnp.float32)] * 2,
        scratch_types=[
            pltpu.VMEM((_NCHUNK, _CH), jnp.int32),
            pltpu.VMEM((_NCHUNK, _CH), jnp.int32),
            pltpu.VMEM((_CH, 2 * EMB), jnp.float32),
            pltpu.VMEM((_CH, 2 * EMB), jnp.float32),
            pltpu.SemaphoreType.DMA,
            pltpu.SemaphoreType.DMA,
        ],
        compiler_params=pltpu.CompilerParams(use_tc_tiling_on_sc=False),
    )


def _unpack(x):
    bits = lax.bitcast_convert_type(x, jnp.uint32)
    hi = lax.bitcast_convert_type(bits & jnp.uint32(0xFFFF0000), jnp.float32)
    lo = lax.bitcast_convert_type(bits << 16, jnp.float32)
    return hi, lo


def _mlp_body(gu_ref, gi_ref,
              w1a_ref, w1b_ref, b1_ref, w2_ref, b2_ref, w3_ref, b3_ref,
              wog_ref, woh_ref, bo_ref, out_ref):
    dot = functools.partial(jnp.dot, preferred_element_type=jnp.float32)
    gmf_u, mlp_u = _unpack(gu_ref[:, :EMB])
    gmf_i, mlp_i = _unpack(gi_ref[:, EMB:])
    h = jnp.maximum(dot(mlp_u, w1a_ref[...]) +
                    dot(mlp_i, w1b_ref[...]) + b1_ref[...], 0.0)
    h = jnp.maximum(dot(h, w2_ref[...]) + b2_ref[...], 0.0)
    h = jnp.maximum(dot(h, w3_ref[...]) + b3_ref[...], 0.0)
    gmf = gmf_u * gmf_i
    out_ref[...] = dot(gmf, wog_ref[...]) + dot(h, woh_ref[...]) + bo_ref[...]


def _mlp(gu, gi, w1a, w1b, b1, w2, b2, w3, b3, wog, woh, bo):
    grid = BATCH // _BB
    row = lambda i: (i, 0)
    rep = lambda i: (0, 0)
    emb_spec = pl.BlockSpec((_BB, 2 * EMB), row)
    full = lambda a: pl.BlockSpec(a.shape, rep)
    return pl.pallas_call(
        _mlp_body,
        grid=(grid,),
        in_specs=[emb_spec, emb_spec,
                  full(w1a), full(w1b), full(b1), full(w2), full(b2),
                  full(w3), full(b3), full(wog), full(woh), full(bo)],
        out_specs=pl.BlockSpec((_BB, 1), row),
        out_shape=jax.ShapeDtypeStruct((BATCH, 1), jnp.float32),
        compiler_params=pltpu.CompilerParams(
            dimension_semantics=("arbitrary",)),
    )(gu, gi, w1a, w1b, b1, w2, b2, w3, b3, wog, woh, bo)


def kernel(user, item, gmf_user, gmf_item, mlp_user, mlp_item,
           W1, b1, W2, b2, W3, b3, Wo, bo):
    user2d = user.astype(jnp.int32).reshape(BATCH // _CH, _CH)
    item2d = item.astype(jnp.int32).reshape(BATCH // _CH, _CH)
    p = _pack(gmf_user.T, mlp_user.T, gmf_item.T, mlp_item.T)
    g_u, g_i = _sc_gather()(user2d, item2d, p)
    out = _mlp(g_u, g_i,
               W1[:EMB], W1[EMB:], b1.reshape(1, HID),
               W2, b2.reshape(1, HID // 2), W3, b3.reshape(1, EMB),
               Wo[:EMB], Wo[EMB:], bo.reshape(1, 1))
    return out.reshape(BATCH)


# speedup vs baseline: 2.1279x; 1.0057x over previous
import functools

import jax
import jax.numpy as jnp
from jax import lax
from jax.experimental import pallas as pl
from jax.experimental.pallas import tpu as pltpu
from jax.experimental.pallas import tpu_sc as plsc

BATCH = 16384
NROWS = 1000000
EMB = 64
HID = 128

_NC = 2
_NS = 16
_NW = _NC * _NS
_RPW = BATCH // _NW
_CH = 128
_NCHUNK = _RPW // _CH

_BT = 12288
_BB = 2048


def _round_bits(x):
    bits = lax.bitcast_convert_type(x, jnp.uint32)
    rounded = bits + 0x7FFF + ((bits >> 16) & 1)
    return rounded & jnp.uint32(0xFFFF0000)


def _pack_body(a_ref, b_ref, c_ref, d_ref, out_ref):
    pair = lambda x_ref, y_ref: (_round_bits(x_ref[...].T) |
                                 (_round_bits(y_ref[...].T) >> 16))
    out_ref[...] = lax.bitcast_convert_type(
        jnp.concatenate([pair(a_ref, b_ref), pair(c_ref, d_ref)], axis=1),
        jnp.float32)


def _pack(tA, tB, tC, tD):
    grid = pl.cdiv(NROWS, _BT)
    tspec = pl.BlockSpec((EMB, _BT), lambda i: (0, i))
    return pl.pallas_call(
        _pack_body,
        grid=(grid,),
        in_specs=[tspec, tspec, tspec, tspec],
        out_specs=pl.BlockSpec((_BT, 2 * EMB), lambda i: (i, 0)),
        out_shape=jax.ShapeDtypeStruct((NROWS, 2 * EMB), jnp.float32),
        compiler_params=pltpu.CompilerParams(
            dimension_semantics=("arbitrary",),
            vmem_limit_bytes=100 * 1024 * 1024),
    )(tA, tB, tC, tD)


def _sc_gather_body(uidx_hbm, iidx_hbm, p_hbm,
                    out_u, out_i,
                    uidx_v, iidx_v, buf_a, buf_b, sem_a, sem_b):
    wid = lax.axis_index("s") * _NC + lax.axis_index("c")
    base = wid * _RPW
    pltpu.sync_copy(uidx_hbm.at[pl.ds(wid * _NCHUNK, _NCHUNK)], uidx_v)
    pltpu.sync_copy(iidx_hbm.at[pl.ds(wid * _NCHUNK, _NCHUNK)], iidx_v)

    bufs = (buf_a, buf_b)
    sems = (sem_a, sem_b)

    for tbl, idxv, out in ((p_hbm, uidx_v, out_u), (p_hbm, iidx_v, out_i)):
        def fire(c):
            return pltpu.async_copy(tbl.at[idxv.at[c]], bufs[c % 2],
                                    sems[c % 2])

        pending = fire(0)
        for c in range(_NCHUNK):
            nxt = fire(c + 1) if c + 1 < _NCHUNK else None
            pending.wait()
            pltpu.sync_copy(bufs[c % 2],
                            out.at[pl.ds(base + c * _CH, _CH)])
            pending = nxt


@functools.cache
def _sc_gather():
    return pl.kernel(
        _sc_gather_body,
        mesh=plsc.VectorSubcoreMesh(core_axis_name="c", subcore_axis_name="s"),
        out_type=[jax.ShapeDtypeStruct((BATCH, 2 * EMB), jnp.float32)] * 2,
        scratch_types=[
            pltpu.VMEM((_NCHUNK, _CH), jnp.int32),
            pltpu.VMEM((_NCHUNK, _CH), jnp.int32),
            pltpu.VMEM((_CH, 2 * EMB), jnp.float32),
            pltpu.VMEM((_CH, 2 * EMB), jnp.float32),
            pltpu.SemaphoreType.DMA,
            pltpu.SemaphoreType.DMA,
        ],
        compiler_params=pltpu.CompilerParams(use_tc_tiling_on_sc=False),
    )


def _unpack(x):
    bits = lax.bitcast_convert_type(x, jnp.uint32)
    hi = lax.bitcast_convert_type(bits & jnp.uint32(0xFFFF0000), jnp.float32)
    lo = lax.bitcast_convert_type(bits << 16, jnp.float32)
    return hi, lo


def _mlp_body(gu_ref, gi_ref,
              w1a_ref, w1b_ref, b1_ref, w2_ref, b2_ref, w3_ref, b3_ref,
              wog_ref, woh_ref, bo_ref, out_ref):
    dot = functools.partial(jnp.dot, preferred_element_type=jnp.float32)
    gmf_u, mlp_u = _unpack(gu_ref[:, :EMB])
    gmf_i, mlp_i = _unpack(gi_ref[:, EMB:])
    h = jnp.maximum(dot(mlp_u, w1a_ref[...]) +
                    dot(mlp_i, w1b_ref[...]) + b1_ref[...], 0.0)
    h = jnp.maximum(dot(h, w2_ref[...]) + b2_ref[...], 0.0)
    h = jnp.maximum(dot(h, w3_ref[...]) + b3_ref[...], 0.0)
    gmf = gmf_u * gmf_i
    out_ref[...] = dot(gmf, wog_ref[...]) + dot(h, woh_ref[...]) + bo_ref[...]


def _mlp(gu, gi, w1a, w1b, b1, w2, b2, w3, b3, wog, woh, bo):
    grid = BATCH // _BB
    row = lambda i: (i, 0)
    rep = lambda i: (0, 0)
    emb_spec = pl.BlockSpec((_BB, 2 * EMB), row)
    full = lambda a: pl.BlockSpec(a.shape, rep)
    return pl.pallas_call(
        _mlp_body,
        grid=(grid,),
        in_specs=[emb_spec, emb_spec,
                  full(w1a), full(w1b), full(b1), full(w2), full(b2),
                  full(w3), full(b3), full(wog), full(woh), full(bo)],
        out_specs=pl.BlockSpec((_BB, 1), row),
        out_shape=jax.ShapeDtypeStruct((BATCH, 1), jnp.float32),
        compiler_params=pltpu.CompilerParams(
            dimension_semantics=("arbitrary",)),
    )(gu, gi, w1a, w1b, b1, w2, b2, w3, b3, wog, woh, bo)


def kernel(user, item, gmf_user, gmf_item, mlp_user, mlp_item,
           W1, b1, W2, b2, W3, b3, Wo, bo):
    user2d = user.astype(jnp.int32).reshape(BATCH // _CH, _CH)
    item2d = item.astype(jnp.int32).reshape(BATCH // _CH, _CH)
    p = _pack(gmf_user.T, mlp_user.T, gmf_item.T, mlp_item.T)
    g_u, g_i = _sc_gather()(user2d, item2d, p)
    out = _mlp(g_u, g_i,
               W1[:EMB], W1[EMB:], b1.reshape(1, HID),
               W2, b2.reshape(1, HID // 2), W3, b3.reshape(1, EMB),
               Wo[:EMB], Wo[EMB:], bo.reshape(1, 1))
    return out.reshape(BATCH)


# speedup vs baseline: 2.2019x; 1.0348x over previous
import functools

import jax
import jax.numpy as jnp
from jax import lax
from jax.experimental import pallas as pl
from jax.experimental.pallas import tpu as pltpu
from jax.experimental.pallas import tpu_sc as plsc

BATCH = 16384
NROWS = 1000000
EMB = 64
HID = 128

_NC = 2
_NS = 16
_NW = _NC * _NS
_RPW = BATCH // _NW
_CH = 128
_NCHUNK = _RPW // _CH

_BT = 12288
_BB = 2048


def _round_bits(x):
    bits = lax.bitcast_convert_type(x, jnp.uint32)
    return (bits + 0x8000) & jnp.uint32(0xFFFF0000)


def _pack_body(a_ref, b_ref, c_ref, d_ref, out_ref):
    pair = lambda x_ref, y_ref: lax.bitcast_convert_type(
        _round_bits(x_ref[...].T) | (_round_bits(y_ref[...].T) >> 16),
        jnp.float32)
    out_ref[:, :EMB] = pair(a_ref, b_ref)
    out_ref[:, EMB:] = pair(c_ref, d_ref)


def _pack(tA, tB, tC, tD):
    grid = pl.cdiv(NROWS, _BT)
    tspec = pl.BlockSpec((EMB, _BT), lambda i: (0, i))
    return pl.pallas_call(
        _pack_body,
        grid=(grid,),
        in_specs=[tspec, tspec, tspec, tspec],
        out_specs=pl.BlockSpec((_BT, 2 * EMB), lambda i: (i, 0)),
        out_shape=jax.ShapeDtypeStruct((NROWS, 2 * EMB), jnp.float32),
        compiler_params=pltpu.CompilerParams(
            dimension_semantics=("arbitrary",),
            vmem_limit_bytes=100 * 1024 * 1024),
    )(tA, tB, tC, tD)


def _sc_gather_body(uidx_hbm, iidx_hbm, p_hbm,
                    out_u, out_i,
                    uidx_v, iidx_v, buf_a, buf_b, sem_a, sem_b):
    wid = lax.axis_index("s") * _NC + lax.axis_index("c")
    base = wid * _RPW
    pltpu.sync_copy(uidx_hbm.at[pl.ds(wid * _NCHUNK, _NCHUNK)], uidx_v)
    pltpu.sync_copy(iidx_hbm.at[pl.ds(wid * _NCHUNK, _NCHUNK)], iidx_v)

    bufs = (buf_a, buf_b)
    sems = (sem_a, sem_b)

    for tbl, idxv, out in ((p_hbm, uidx_v, out_u), (p_hbm, iidx_v, out_i)):
        def fire(c):
            return pltpu.async_copy(tbl.at[idxv.at[c]], bufs[c % 2],
                                    sems[c % 2])

        pending = fire(0)
        for c in range(_NCHUNK):
            nxt = fire(c + 1) if c + 1 < _NCHUNK else None
            pending.wait()
            pltpu.sync_copy(bufs[c % 2],
                            out.at[pl.ds(base + c * _CH, _CH)])
            pending = nxt


@functools.cache
def _sc_gather():
    return pl.kernel(
        _sc_gather_body,
        mesh=plsc.VectorSubcoreMesh(core_axis_name="c", subcore_axis_name="s"),
        out_type=[jax.ShapeDtypeStruct((BATCH, 2 * EMB), jnp.float32)] * 2,
        scratch_types=[
            pltpu.VMEM((_NCHUNK, _CH), jnp.int32),
            pltpu.VMEM((_NCHUNK, _CH), jnp.int32),
            pltpu.VMEM((_CH, 2 * EMB), jnp.float32),
            pltpu.VMEM((_CH, 2 * EMB), jnp.float32),
            pltpu.SemaphoreType.DMA,
            pltpu.SemaphoreType.DMA,
        ],
        compiler_params=pltpu.CompilerParams(use_tc_tiling_on_sc=False),
    )


def _unpack(x):
    bits = lax.bitcast_convert_type(x, jnp.uint32)
    hi = lax.bitcast_convert_type(bits & jnp.uint32(0xFFFF0000), jnp.float32)
    lo = lax.bitcast_convert_type(bits << 16, jnp.float32)
    return hi, lo


def _mlp_body(gu_ref, gi_ref,
              w1a_ref, w1b_ref, b1_ref, w2_ref, b2_ref, w3_ref, b3_ref,
              wog_ref, woh_ref, bo_ref, out_ref):
    dot = functools.partial(jnp.dot, preferred_element_type=jnp.float32)
    gmf_u, mlp_u = _unpack(gu_ref[:, :EMB])
    gmf_i, mlp_i = _unpack(gi_ref[:, EMB:])
    h = jnp.maximum(dot(mlp_u, w1a_ref[...]) +
                    dot(mlp_i, w1b_ref[...]) + b1_ref[...], 0.0)
    h = jnp.maximum(dot(h, w2_ref[...]) + b2_ref[...], 0.0)
    h = jnp.maximum(dot(h, w3_ref[...]) + b3_ref[...], 0.0)
    gmf = gmf_u * gmf_i
    out_ref[...] = dot(gmf, wog_ref[...]) + dot(h, woh_ref[...]) + bo_ref[...]


def _mlp(gu, gi, w1a, w1b, b1, w2, b2, w3, b3, wog, woh, bo):
    grid = BATCH // _BB
    row = lambda i: (i, 0)
    rep = lambda i: (0, 0)
    emb_spec = pl.BlockSpec((_BB, 2 * EMB), row)
    full = lambda a: pl.BlockSpec(a.shape, rep)
    return pl.pallas_call(
        _mlp_body,
        grid=(grid,),
        in_specs=[emb_spec, emb_spec,
                  full(w1a), full(w1b), full(b1), full(w2), full(b2),
                  full(w3), full(b3), full(wog), full(woh), full(bo)],
        out_specs=pl.BlockSpec((_BB, 1), row),
        out_shape=jax.ShapeDtypeStruct((BATCH, 1), jnp.float32),
        compiler_params=pltpu.CompilerParams(
            dimension_semantics=("arbitrary",)),
    )(gu, gi, w1a, w1b, b1, w2, b2, w3, b3, wog, woh, bo)


def kernel(user, item, gmf_user, gmf_item, mlp_user, mlp_item,
           W1, b1, W2, b2, W3, b3, Wo, bo):
    user2d = user.astype(jnp.int32).reshape(BATCH // _CH, _CH)
    item2d = item.astype(jnp.int32).reshape(BATCH // _CH, _CH)
    p = _pack(gmf_user.T, mlp_user.T, gmf_item.T, mlp_item.T)
    g_u, g_i = _sc_gather()(user2d, item2d, p)
    out = _mlp(g_u, g_i,
               W1[:EMB], W1[EMB:], b1.reshape(1, HID),
               W2, b2.reshape(1, HID // 2), W3, b3.reshape(1, EMB),
               Wo[:EMB], Wo[EMB:], bo.reshape(1, 1))
    return out.reshape(BATCH)
